# Initial kernel scaffold; baseline (speedup 1.0000x reference)
import jax
import jax.numpy as jnp
from jax.experimental import pallas as pl


def kernel(cat_seq, num_seq, emb_strikeId, emb_handId, emb_strengthId, emb_spinId, emb_pointId, emb_actionId, emb_positionId, num_W, num_b, proj_W, proj_b, ln_g, ln_b):
    raise NotImplementedError("write your pallas kernel here")


# speedup vs baseline: 8.3553x; 8.3553x over previous
import functools

import jax
import jax.numpy as jnp
from jax.experimental import pallas as pl
from jax.experimental.pallas import tpu as pltpu

_N_CLS = (5, 3, 4, 6, 10, 19, 4)
_E_DIM = (16, 8, 8, 16, 24, 32, 8)
_ROW_OFF = (0, 5, 8, 12, 18, 28, 47)
_COL_OFF = (0, 16, 24, 32, 48, 72, 104)
_NUM_ROW = 51
_ONE_ROW = 52
_K = 64
_D_IN = 128
_D_MODEL = 256


def _fused_kernel(idx_ref, num_ref, t_ref, w_ref, pb_ref, g_ref, b_ref,
                  out_ref, a_ref):
    @pl.when(pl.program_id(0) == 0)
    def _():
        a = jnp.dot(t_ref[...], w_ref[...], preferred_element_type=jnp.float32)
        row = jax.lax.broadcasted_iota(jnp.int32, (_K, 1), 0)
        a_ref[...] = a + jnp.where(row == _ONE_ROW, 1.0, 0.0) * pb_ref[...]

    rows = idx_ref.shape[0]
    lane = jax.lax.broadcasted_iota(jnp.int32, (rows, _K), 1)
    m = jnp.zeros((rows, _K), jnp.float32)
    for i in range(7):
        m += (lane == (idx_ref[:, i:i + 1] + _ROW_OFF[i])).astype(jnp.float32)
    m += jnp.where(lane == _NUM_ROW, 1.0, 0.0) * num_ref[...]
    m += jnp.where(lane == _ONE_ROW, 1.0, 0.0)

    h = jnp.dot(m, a_ref[...], preferred_element_type=jnp.float32)
    h = 0.5 * h * (1.0 + jax.lax.erf(h * 0.7071067811865476))
    mu = jnp.mean(h, axis=1, keepdims=True)
    d = h - mu
    var = jnp.mean(d * d, axis=1, keepdims=True)
    out_ref[...] = d * jax.lax.rsqrt(var + 1e-5) * g_ref[...] + b_ref[...]


@functools.partial(jax.jit, static_argnames=())
def _run(cat_seq, num_seq, tables, num_W, num_b, proj_W, proj_b, ln_g, ln_b):
    B, L, _ = cat_seq.shape
    n = B * L
    idx = cat_seq.reshape(n, 7).astype(jnp.int32)
    num = num_seq.reshape(n, 1)

    t = jnp.zeros((_K, _D_IN), jnp.float32)
    for i in range(7):
        t = jax.lax.dynamic_update_slice(
            t, tables[i], (_ROW_OFF[i], _COL_OFF[i]))
    t = jax.lax.dynamic_update_slice(t, num_W.reshape(1, 16), (_NUM_ROW, 112))
    t = jax.lax.dynamic_update_slice(t, num_b.reshape(1, 16), (_ONE_ROW, 112))

    block = 2048
    grid = (n // block,)
    out = pl.pallas_call(
        _fused_kernel,
        grid=grid,
        in_specs=[
            pl.BlockSpec((block, 7), lambda i: (i, 0)),
            pl.BlockSpec((block, 1), lambda i: (i, 0)),
            pl.BlockSpec((_K, _D_IN), lambda i: (0, 0)),
            pl.BlockSpec((_D_IN, _D_MODEL), lambda i: (0, 0)),
            pl.BlockSpec((1, _D_MODEL), lambda i: (0, 0)),
            pl.BlockSpec((1, _D_MODEL), lambda i: (0, 0)),
            pl.BlockSpec((1, _D_MODEL), lambda i: (0, 0)),
        ],
        out_specs=pl.BlockSpec((block, _D_MODEL), lambda i: (i, 0)),
        out_shape=jax.ShapeDtypeStruct((n, _D_MODEL), jnp.float32),
        scratch_shapes=[pltpu.VMEM((_K, _D_MODEL), jnp.float32)],
        compiler_params=pltpu.CompilerParams(
            dimension_semantics=("arbitrary",)),
    )(idx, num, t, proj_W, proj_b.reshape(1, -1), ln_g.reshape(1, -1),
      ln_b.reshape(1, -1))
    return out.reshape(B, L, _D_MODEL)


def kernel(cat_seq, num_seq, emb_strikeId, emb_handId, emb_strengthId,
           emb_spinId, emb_pointId, emb_actionId, emb_positionId,
           num_W, num_b, proj_W, proj_b, ln_g, ln_b):
    tables = (emb_strikeId, emb_handId, emb_strengthId, emb_spinId,
              emb_pointId, emb_actionId, emb_positionId)
    return _run(cat_seq, num_seq, tables, num_W, num_b, proj_W, proj_b,
                ln_g, ln_b)


# speedup vs baseline: 11.4232x; 1.3672x over previous
import functools

import jax
import jax.numpy as jnp
import numpy as np
from jax.experimental import pallas as pl
from jax.experimental.pallas import tpu as pltpu

_N_CLS = (5, 3, 4, 6, 10, 19, 4)
_E_DIM = (16, 8, 8, 16, 24, 32, 8)
_ROW_OFF = (0, 5, 8, 12, 18, 28, 47)
_COL_OFF = (0, 16, 24, 32, 48, 72, 104)
_NUM_ROW = 51
_ONE_ROW = 52
_K = 64
_D_IN = 128
_D_MODEL = 256
_L = 50


def _sel_consts():
    s = np.zeros((7, _K), np.float32)
    base = np.full((1, _K), -1000.0, np.float32)
    for f in range(7):
        lo, hi = _ROW_OFF[f], _ROW_OFF[f] + _N_CLS[f]
        s[f, lo:hi] = 1.0
        base[0, lo:hi] = _ROW_OFF[f]
    base[0, _ONE_ROW] = _ONE_ROW
    return jnp.asarray(s), jnp.asarray(base)


def _fused_kernel(idx_ref, num_ref, t_ref, w_ref, pb_ref, s_ref, base_ref,
                  g_ref, b_ref, out_ref, a_ref):
    @pl.when(pl.program_id(0) == 0)
    def _():
        a = jnp.dot(t_ref[...], w_ref[...], preferred_element_type=jnp.float32)
        row = jax.lax.broadcasted_iota(jnp.int32, (_K, 1), 0)
        a_ref[...] = a + jnp.where(row == _ONE_ROW, 1.0, 0.0) * pb_ref[...]

    bb = idx_ref.shape[0]
    lane = jax.lax.broadcasted_iota(jnp.int32, (_L, _K), 1).astype(jnp.float32)
    for j in range(bb):
        idx = idx_ref[j].astype(jnp.float32)
        s = jnp.dot(idx, s_ref[...],
                    preferred_element_type=jnp.float32) + base_ref[...]
        m = (s == lane).astype(jnp.float32)
        h = jnp.dot(m, a_ref[...], preferred_element_type=jnp.float32)
        h += jnp.dot(num_ref[j], a_ref[_NUM_ROW:_NUM_ROW + 1, :],
                     preferred_element_type=jnp.float32)
        h = 0.5 * h * (1.0 + jax.lax.erf(h * 0.7071067811865476))
        mu = jnp.mean(h, axis=1, keepdims=True)
        d = h - mu
        var = jnp.mean(d * d, axis=1, keepdims=True)
        out_ref[j] = d * jax.lax.rsqrt(var + 1e-5) * g_ref[...] + b_ref[...]


@jax.jit
def _run(cat_seq, num_seq, tables, num_W, num_b, proj_W, proj_b, ln_g, ln_b):
    B, L, _ = cat_seq.shape
    idx = cat_seq.astype(jnp.int32)

    t = jnp.zeros((_K, _D_IN), jnp.float32)
    for i in range(7):
        t = jax.lax.dynamic_update_slice(
            t, tables[i], (_ROW_OFF[i], _COL_OFF[i]))
    t = jax.lax.dynamic_update_slice(t, num_W.reshape(1, 16), (_NUM_ROW, 112))
    t = jax.lax.dynamic_update_slice(t, num_b.reshape(1, 16), (_ONE_ROW, 112))
    s_mat, base = _sel_consts()

    bb = 32
    grid = (B // bb,)
    out = pl.pallas_call(
        _fused_kernel,
        grid=grid,
        in_specs=[
            pl.BlockSpec((bb, L, 7), lambda i: (i, 0, 0)),
            pl.BlockSpec((bb, L, 1), lambda i: (i, 0, 0)),
            pl.BlockSpec((_K, _D_IN), lambda i: (0, 0)),
            pl.BlockSpec((_D_IN, _D_MODEL), lambda i: (0, 0)),
            pl.BlockSpec((1, _D_MODEL), lambda i: (0, 0)),
            pl.BlockSpec((7, _K), lambda i: (0, 0)),
            pl.BlockSpec((1, _K), lambda i: (0, 0)),
            pl.BlockSpec((1, _D_MODEL), lambda i: (0, 0)),
            pl.BlockSpec((1, _D_MODEL), lambda i: (0, 0)),
        ],
        out_specs=pl.BlockSpec((bb, L, _D_MODEL), lambda i: (i, 0, 0)),
        out_shape=jax.ShapeDtypeStruct((B, L, _D_MODEL), jnp.float32),
        scratch_shapes=[pltpu.VMEM((_K, _D_MODEL), jnp.float32)],
        compiler_params=pltpu.CompilerParams(
            dimension_semantics=("arbitrary",)),
    )(idx, num_seq, t, proj_W, proj_b.reshape(1, -1), s_mat, base,
      ln_g.reshape(1, -1), ln_b.reshape(1, -1))
    return out


def kernel(cat_seq, num_seq, emb_strikeId, emb_handId, emb_strengthId,
           emb_spinId, emb_pointId, emb_actionId, emb_positionId,
           num_W, num_b, proj_W, proj_b, ln_g, ln_b):
    tables = (emb_strikeId, emb_handId, emb_strengthId, emb_spinId,
              emb_pointId, emb_actionId, emb_positionId)
    return _run(cat_seq, num_seq, tables, num_W, num_b, proj_W, proj_b,
                ln_g, ln_b)


# speedup vs baseline: 18.0003x; 1.5758x over previous
import jax
import jax.numpy as jnp
import numpy as np
from jax.experimental import pallas as pl
from jax.experimental.pallas import tpu as pltpu

_N_CLS = (5, 3, 4, 6, 10, 19, 4)
_E_DIM = (16, 8, 8, 16, 24, 32, 8)
_ROW_OFF = (0, 5, 8, 12, 18, 28, 47)
_COL_OFF = (0, 16, 24, 32, 48, 72, 104)
_NUM_ROW = 51
_ONE_ROW = 52
_K = 64
_D_IN = 128
_D_MODEL = 256
_L = 50
_BB = 256


def _sel_consts():
    s = np.zeros((8, _K), np.float32)
    base = np.full((1, _K), -1000.0, np.float32)
    for f in range(7):
        lo, hi = _ROW_OFF[f], _ROW_OFF[f] + _N_CLS[f]
        s[f, lo:hi] = 1.0
        base[0, lo:hi] = _ROW_OFF[f]
    base[0, _ONE_ROW] = _ONE_ROW
    return jnp.asarray(s), jnp.asarray(base)


def _fused_kernel(x_ref, t_ref, w_ref, pb_ref, s_ref, base_ref,
                  g_ref, b_ref, out_ref, a_ref):
    @pl.when(pl.program_id(0) == 0)
    def _():
        a = jnp.dot(t_ref[...], w_ref[...], preferred_element_type=jnp.float32)
        row = jax.lax.broadcasted_iota(jnp.int32, (_K, 1), 0)
        a_ref[...] = a + jnp.where(row == _ONE_ROW, 1.0, 0.0) * pb_ref[...]

    lane = jax.lax.broadcasted_iota(
        jnp.int32, (_BB, _K), 1).astype(jnp.float32)
    for l in range(_L):
        x = x_ref[:, 8 * l:8 * l + 8]
        s = jnp.dot(x, s_ref[...],
                    preferred_element_type=jnp.float32) + base_ref[...]
        m = (s == lane).astype(jnp.float32)
        h = jnp.dot(m, a_ref[...], preferred_element_type=jnp.float32)
        h += jnp.dot(x[:, 7:8], a_ref[_NUM_ROW:_NUM_ROW + 1, :],
                     preferred_element_type=jnp.float32)
        h = 0.5 * h * (1.0 + jax.lax.erf(h * 0.7071067811865476))
        mu = jnp.mean(h, axis=1, keepdims=True)
        d = h - mu
        var = jnp.mean(d * d, axis=1, keepdims=True)
        out_ref[l] = d * jax.lax.rsqrt(var + 1e-5) * g_ref[...] + b_ref[...]


@jax.jit
def _run(cat_seq, num_seq, tables, num_W, num_b, proj_W, proj_b, ln_g, ln_b):
    B, L, _ = cat_seq.shape

    x8 = jnp.concatenate([cat_seq.astype(jnp.float32), num_seq], axis=2)
    x8 = x8.reshape(B, L * 8)

    t = jnp.zeros((_K, _D_IN), jnp.float32)
    for i in range(7):
        t = jax.lax.dynamic_update_slice(
            t, tables[i], (_ROW_OFF[i], _COL_OFF[i]))
    t = jax.lax.dynamic_update_slice(t, num_W.reshape(1, 16), (_NUM_ROW, 112))
    t = jax.lax.dynamic_update_slice(t, num_b.reshape(1, 16), (_ONE_ROW, 112))
    s_mat, base = _sel_consts()

    grid = (B // _BB,)
    out = pl.pallas_call(
        _fused_kernel,
        grid=grid,
        in_specs=[
            pl.BlockSpec((_BB, L * 8), lambda i: (i, 0)),
            pl.BlockSpec((_K, _D_IN), lambda i: (0, 0)),
            pl.BlockSpec((_D_IN, _D_MODEL), lambda i: (0, 0)),
            pl.BlockSpec((1, _D_MODEL), lambda i: (0, 0)),
            pl.BlockSpec((8, _K), lambda i: (0, 0)),
            pl.BlockSpec((1, _K), lambda i: (0, 0)),
            pl.BlockSpec((1, _D_MODEL), lambda i: (0, 0)),
            pl.BlockSpec((1, _D_MODEL), lambda i: (0, 0)),
        ],
        out_specs=pl.BlockSpec((L, _BB, _D_MODEL), lambda i: (0, i, 0)),
        out_shape=jax.ShapeDtypeStruct((L, B, _D_MODEL), jnp.float32),
        scratch_shapes=[pltpu.VMEM((_K, _D_MODEL), jnp.float32)],
        compiler_params=pltpu.CompilerParams(
            dimension_semantics=("arbitrary",)),
    )(x8, t, proj_W, proj_b.reshape(1, -1), s_mat, base,
      ln_g.reshape(1, -1), ln_b.reshape(1, -1))
    return jnp.transpose(out, (1, 0, 2))


def kernel(cat_seq, num_seq, emb_strikeId, emb_handId, emb_strengthId,
           emb_spinId, emb_pointId, emb_actionId, emb_positionId,
           num_W, num_b, proj_W, proj_b, ln_g, ln_b):
    tables = (emb_strikeId, emb_handId, emb_strengthId, emb_spinId,
              emb_pointId, emb_actionId, emb_positionId)
    return _run(cat_seq, num_seq, tables, num_W, num_b, proj_W, proj_b,
                ln_g, ln_b)
